# Initial kernel scaffold; baseline (speedup 1.0000x reference)
import jax
import jax.numpy as jnp
from jax.experimental import pallas as pl


def kernel(x, edge_index, params):
    raise NotImplementedError("write your pallas kernel here")


# speedup vs baseline: 7.8732x; 7.8732x over previous
import functools

import jax
import jax.numpy as jnp
from jax import lax
from jax.experimental import pallas as pl
from jax.experimental.pallas import tpu as pltpu
from jax.experimental.pallas import tpu_sc as plsc

N = 10000
E = 320000
F_IN = 128
H = 64
NC = 2
NS = 16
NW = NC * NS
EP = E // NW
C = 80
G = EP // C
N_PAD = 10240
NROWS = N_PAD // NS
DEGW = 16


def _make_sc_agg(with_deg: bool):
    mesh = plsc.VectorSubcoreMesh(core_axis_name="c", subcore_axis_name="s")
    out_type = [jax.ShapeDtypeStruct((NC, N_PAD, H), jnp.float32)]
    if with_deg:
        out_type.append(jax.ShapeDtypeStruct((NC, N_PAD, DEGW), jnp.float32))
    scratch = [
        pltpu.VMEM_SHARED((N_PAD, H), jnp.float32),
        pltpu.VMEM((G, C), jnp.int32),
        pltpu.VMEM((G, C), jnp.int32),
        pltpu.VMEM((C, H), jnp.float32),
        pltpu.SemaphoreType.DMA,
    ]
    if with_deg:
        scratch += [
            pltpu.VMEM_SHARED((N_PAD, DEGW), jnp.float32),
            pltpu.VMEM((C, DEGW), jnp.float32),
        ]

    def body(y_hbm, row_hbm, col_hbm, z_h_hbm, z_d_hbm, ones_hbm,
             acc_out, *rest):
        if with_deg:
            deg_out, acc_sh, row_v, col_v, gbuf, sem, deg_sh, ones_v = rest
        else:
            acc_sh, row_v, col_v, gbuf, sem = rest
        cid = lax.axis_index("c")
        sid = lax.axis_index("s")
        wid = sid * NC + cid
        base_r = sid * NROWS

        pltpu.sync_copy(z_h_hbm.at[pl.ds(base_r, NROWS)],
                        acc_sh.at[pl.ds(base_r, NROWS)])
        if with_deg:
            pltpu.sync_copy(z_d_hbm.at[pl.ds(base_r, NROWS)],
                            deg_sh.at[pl.ds(base_r, NROWS)])
            pltpu.sync_copy(ones_hbm, ones_v)
        pltpu.sync_copy(row_hbm.at[wid], row_v)
        pltpu.sync_copy(col_hbm.at[wid], col_v)
        plsc.subcore_barrier()

        def chunk(g, carry):
            pltpu.async_copy(y_hbm.at[col_v.at[g]], gbuf, sem).wait()
            pltpu.sync_copy(gbuf, acc_sh.at[row_v.at[g]], add=True)
            if with_deg:
                pltpu.sync_copy(ones_v, deg_sh.at[row_v.at[g]], add=True)
            return carry

        lax.fori_loop(0, G, chunk, 0)
        plsc.subcore_barrier()

        pltpu.sync_copy(acc_sh.at[pl.ds(base_r, NROWS)],
                        acc_out.at[cid, pl.ds(base_r, NROWS)])
        if with_deg:
            pltpu.sync_copy(deg_sh.at[pl.ds(base_r, NROWS)],
                            deg_out.at[cid, pl.ds(base_r, NROWS)])

    return pl.kernel(body, out_type=tuple(out_type), mesh=mesh,
                     scratch_types=scratch,
                     compiler_params=pltpu.CompilerParams(
                         use_tc_tiling_on_sc=False))


_sc_agg_deg = _make_sc_agg(with_deg=True)
_sc_agg = _make_sc_agg(with_deg=False)


def _mm(a, w):
    return lax.dot_general(a, w, (((1,), (1,)), ((), ())),
                           preferred_element_type=jnp.float32)


def _tc_pre_body(x_ref, wn_ref, ws_ref, y_ref, s_ref):
    x = x_ref[...]
    y_ref[...] = _mm(x, wn_ref[...])
    s_ref[...] = _mm(x, ws_ref[...])


_tc_pre = pl.pallas_call(
    _tc_pre_body,
    out_shape=(jax.ShapeDtypeStruct((N, H), jnp.float32),
               jax.ShapeDtypeStruct((N, H), jnp.float32)),
)


def _combine_ln(s, p_ref, inv, bs, bn, g, beta):
    neigh = (p_ref[0, :N, :] + p_ref[1, :N, :]) * inv
    z = jax.nn.relu(s + bs + neigh + bn)
    m = jnp.mean(z, axis=-1, keepdims=True)
    v = jnp.mean((z - m) * (z - m), axis=-1, keepdims=True)
    return (z - m) * lax.rsqrt(v + 1e-5) * g + beta


def _tc_mid1_body(s_ref, p_ref, dp_ref, bs_ref, bn_ref, g_ref, beta_ref,
                  wn_ref, ws_ref, y_ref, s2_ref, inv_ref):
    deg = dp_ref[0, :N, 0:1] + dp_ref[1, :N, 0:1]
    inv = 1.0 / jnp.maximum(deg, 1.0)
    inv_ref[...] = inv
    x = _combine_ln(s_ref[...], p_ref, inv, bs_ref[...], bn_ref[...],
                    g_ref[...], beta_ref[...])
    y_ref[...] = _mm(x, wn_ref[...])
    s2_ref[...] = _mm(x, ws_ref[...])


_tc_mid1 = pl.pallas_call(
    _tc_mid1_body,
    out_shape=(jax.ShapeDtypeStruct((N, H), jnp.float32),
               jax.ShapeDtypeStruct((N, H), jnp.float32),
               jax.ShapeDtypeStruct((N, 1), jnp.float32)),
)


def _tc_mid_body(s_ref, p_ref, inv_ref, bs_ref, bn_ref, g_ref, beta_ref,
                 wn_ref, ws_ref, y_ref, s2_ref):
    x = _combine_ln(s_ref[...], p_ref, inv_ref[...], bs_ref[...],
                    bn_ref[...], g_ref[...], beta_ref[...])
    y_ref[...] = _mm(x, wn_ref[...])
    s2_ref[...] = _mm(x, ws_ref[...])


_tc_mid = pl.pallas_call(
    _tc_mid_body,
    out_shape=(jax.ShapeDtypeStruct((N, H), jnp.float32),
               jax.ShapeDtypeStruct((N, H), jnp.float32)),
)


def _tc_final_body(s_ref, p_ref, inv_ref, bs_ref, bn_ref, g_ref, beta_ref,
                   hw1_ref, hb1_ref, hw2_ref, hb2_ref, out_ref):
    x = _combine_ln(s_ref[...], p_ref, inv_ref[...], bs_ref[...],
                    bn_ref[...], g_ref[...], beta_ref[...])
    h = jax.nn.relu(_mm(x, hw1_ref[...]) + hb1_ref[...])
    out_ref[...] = jnp.sum(h * hw2_ref[...], axis=-1, keepdims=True) + hb2_ref[0, 0]


_tc_final = pl.pallas_call(
    _tc_final_body,
    out_shape=jax.ShapeDtypeStruct((N, 1), jnp.float32),
)


def kernel(x, edge_index, params):
    row = edge_index[0].reshape(NW, G, C)
    col = edge_index[1].reshape(NW, G, C)
    z_h = jnp.zeros((N_PAD, H), jnp.float32)
    z_d = jnp.zeros((N_PAD, DEGW), jnp.float32)
    ones = jnp.ones((C, DEGW), jnp.float32)
    Ws, bs, Wn, bn = params["Ws"], params["bs"], params["Wn"], params["bn"]
    g, beta = params["g"], params["beta"]
    b2 = lambda v: v.reshape(1, -1)

    y1, s1 = _tc_pre(x, Wn[0], Ws[0])
    acc1, degp = _sc_agg_deg(y1, row, col, z_h, z_d, ones)
    y2, s2, inv = _tc_mid1(s1, acc1, degp, b2(bs[0]), b2(bn[0]), b2(g[0]),
                           b2(beta[0]), Wn[1], Ws[1])
    (acc2,) = _sc_agg(y2, row, col, z_h, z_d, ones)
    y3, s3 = _tc_mid(s2, acc2, inv, b2(bs[1]), b2(bn[1]), b2(g[1]),
                     b2(beta[1]), Wn[2], Ws[2])
    (acc3,) = _sc_agg(y3, row, col, z_h, z_d, ones)
    out = _tc_final(s3, acc3, inv, b2(bs[2]), b2(bn[2]), b2(g[2]),
                    b2(beta[2]), params["Hw1"], b2(params["Hb1"]),
                    params["Hw2"], b2(params["Hb2"]))
    return out.reshape(N)


# speedup vs baseline: 11.8224x; 1.5016x over previous
import functools

import jax
import jax.numpy as jnp
from jax import lax
from jax.experimental import pallas as pl
from jax.experimental.pallas import tpu as pltpu
from jax.experimental.pallas import tpu_sc as plsc

N = 10000
E = 320000
F_IN = 128
H = 64
NC = 2
NS = 16
NW = NC * NS
EP = E // NW
C = 80
G = EP // C
N_PAD = 10240
NROWS = N_PAD // NS
DEGW = 16


def _make_sc_agg(with_deg: bool):
    mesh = plsc.VectorSubcoreMesh(core_axis_name="c", subcore_axis_name="s")
    out_type = [jax.ShapeDtypeStruct((NC, N_PAD, H), jnp.float32)]
    if with_deg:
        out_type.append(jax.ShapeDtypeStruct((NC, N_PAD, DEGW), jnp.float32))
    scratch = [
        pltpu.VMEM_SHARED((N_PAD, H), jnp.float32),
        pltpu.VMEM((G, C), jnp.int32),
        pltpu.VMEM((G, C), jnp.int32),
        pltpu.VMEM((C, H), jnp.float32),
        pltpu.VMEM((C, H), jnp.float32),
        pltpu.SemaphoreType.DMA,
        pltpu.SemaphoreType.DMA,
    ]
    if with_deg:
        scratch += [
            pltpu.VMEM_SHARED((N_PAD, DEGW), jnp.float32),
            pltpu.VMEM((C, DEGW), jnp.float32),
        ]

    def body(y_hbm, row_hbm, col_hbm, z_h_hbm, z_d_hbm, ones_hbm,
             acc_out, *rest):
        if with_deg:
            (deg_out, acc_sh, row_v, col_v, gbuf0, gbuf1, sem0, sem1,
             deg_sh, ones_v) = rest
        else:
            acc_sh, row_v, col_v, gbuf0, gbuf1, sem0, sem1 = rest
        bufs = (gbuf0, gbuf1)
        sems = (sem0, sem1)
        cid = lax.axis_index("c")
        sid = lax.axis_index("s")
        wid = sid * NC + cid
        base_r = sid * NROWS

        pltpu.sync_copy(z_h_hbm.at[pl.ds(base_r, NROWS)],
                        acc_sh.at[pl.ds(base_r, NROWS)])
        if with_deg:
            pltpu.sync_copy(z_d_hbm.at[pl.ds(base_r, NROWS)],
                            deg_sh.at[pl.ds(base_r, NROWS)])
            pltpu.sync_copy(ones_hbm, ones_v)
        pltpu.sync_copy(row_hbm.at[wid], row_v)
        pltpu.sync_copy(col_hbm.at[wid], col_v)
        plsc.subcore_barrier()

        def fire(g, b):
            pltpu.async_copy(y_hbm.at[col_v.at[g]], bufs[b], sems[b])

        def drain_and_scatter(g, b):
            pltpu.make_async_copy(y_hbm.at[col_v.at[g]], bufs[b],
                                  sems[b]).wait()
            pltpu.sync_copy(bufs[b], acc_sh.at[row_v.at[g]], add=True)
            if with_deg:
                pltpu.sync_copy(ones_v, deg_sh.at[row_v.at[g]], add=True)

        fire(0, 0)
        fire(1, 1)

        def pair(i, carry):
            for b in range(2):
                g = 2 * i + b
                drain_and_scatter(g, b)

                @pl.when(g + 2 < G)
                def _():
                    fire(g + 2, b)
            return carry

        lax.fori_loop(0, (G - 1) // 2, pair, 0)
        drain_and_scatter(G - 1, (G - 1) % 2)
        plsc.subcore_barrier()

        pltpu.sync_copy(acc_sh.at[pl.ds(base_r, NROWS)],
                        acc_out.at[cid, pl.ds(base_r, NROWS)])
        if with_deg:
            pltpu.sync_copy(deg_sh.at[pl.ds(base_r, NROWS)],
                            deg_out.at[cid, pl.ds(base_r, NROWS)])

    return pl.kernel(body, out_type=tuple(out_type), mesh=mesh,
                     scratch_types=scratch,
                     compiler_params=pltpu.CompilerParams(
                         use_tc_tiling_on_sc=False))


_sc_agg_deg = _make_sc_agg(with_deg=True)
_sc_agg = _make_sc_agg(with_deg=False)


def _mm(a, w):
    return lax.dot_general(a, w, (((1,), (1,)), ((), ())),
                           preferred_element_type=jnp.float32)


def _tc_pre_body(x_ref, wn_ref, ws_ref, y_ref, s_ref):
    x = x_ref[...]
    y_ref[...] = _mm(x, wn_ref[...])
    s_ref[...] = _mm(x, ws_ref[...])


_tc_pre = pl.pallas_call(
    _tc_pre_body,
    out_shape=(jax.ShapeDtypeStruct((N, H), jnp.float32),
               jax.ShapeDtypeStruct((N, H), jnp.float32)),
)


def _combine_ln(s, p_ref, inv, bs, bn, g, beta):
    neigh = (p_ref[0, :N, :] + p_ref[1, :N, :]) * inv
    z = jax.nn.relu(s + bs + neigh + bn)
    m = jnp.mean(z, axis=-1, keepdims=True)
    v = jnp.mean((z - m) * (z - m), axis=-1, keepdims=True)
    return (z - m) * lax.rsqrt(v + 1e-5) * g + beta


def _tc_mid1_body(s_ref, p_ref, dp_ref, bs_ref, bn_ref, g_ref, beta_ref,
                  wn_ref, ws_ref, y_ref, s2_ref, inv_ref):
    deg = dp_ref[0, :N, 0:1] + dp_ref[1, :N, 0:1]
    inv = 1.0 / jnp.maximum(deg, 1.0)
    inv_ref[...] = inv
    x = _combine_ln(s_ref[...], p_ref, inv, bs_ref[...], bn_ref[...],
                    g_ref[...], beta_ref[...])
    y_ref[...] = _mm(x, wn_ref[...])
    s2_ref[...] = _mm(x, ws_ref[...])


_tc_mid1 = pl.pallas_call(
    _tc_mid1_body,
    out_shape=(jax.ShapeDtypeStruct((N, H), jnp.float32),
               jax.ShapeDtypeStruct((N, H), jnp.float32),
               jax.ShapeDtypeStruct((N, 1), jnp.float32)),
)


def _tc_mid_body(s_ref, p_ref, inv_ref, bs_ref, bn_ref, g_ref, beta_ref,
                 wn_ref, ws_ref, y_ref, s2_ref):
    x = _combine_ln(s_ref[...], p_ref, inv_ref[...], bs_ref[...],
                    bn_ref[...], g_ref[...], beta_ref[...])
    y_ref[...] = _mm(x, wn_ref[...])
    s2_ref[...] = _mm(x, ws_ref[...])


_tc_mid = pl.pallas_call(
    _tc_mid_body,
    out_shape=(jax.ShapeDtypeStruct((N, H), jnp.float32),
               jax.ShapeDtypeStruct((N, H), jnp.float32)),
)


def _tc_final_body(s_ref, p_ref, inv_ref, bs_ref, bn_ref, g_ref, beta_ref,
                   hw1_ref, hb1_ref, hw2_ref, hb2_ref, out_ref):
    x = _combine_ln(s_ref[...], p_ref, inv_ref[...], bs_ref[...],
                    bn_ref[...], g_ref[...], beta_ref[...])
    h = jax.nn.relu(_mm(x, hw1_ref[...]) + hb1_ref[...])
    out_ref[...] = jnp.sum(h * hw2_ref[...], axis=-1, keepdims=True) + hb2_ref[0, 0]


_tc_final = pl.pallas_call(
    _tc_final_body,
    out_shape=jax.ShapeDtypeStruct((N, 1), jnp.float32),
)


def kernel(x, edge_index, params):
    row = edge_index[0].reshape(NW, G, C)
    col = edge_index[1].reshape(NW, G, C)
    z_h = jnp.zeros((N_PAD, H), jnp.float32)
    z_d = jnp.zeros((N_PAD, DEGW), jnp.float32)
    ones = jnp.ones((C, DEGW), jnp.float32)
    Ws, bs, Wn, bn = params["Ws"], params["bs"], params["Wn"], params["bn"]
    g, beta = params["g"], params["beta"]
    b2 = lambda v: v.reshape(1, -1)

    y1, s1 = _tc_pre(x, Wn[0], Ws[0])
    acc1, degp = _sc_agg_deg(y1, row, col, z_h, z_d, ones)
    y2, s2, inv = _tc_mid1(s1, acc1, degp, b2(bs[0]), b2(bn[0]), b2(g[0]),
                           b2(beta[0]), Wn[1], Ws[1])
    (acc2,) = _sc_agg(y2, row, col, z_h, z_d, ones)
    y3, s3 = _tc_mid(s2, acc2, inv, b2(bs[1]), b2(bn[1]), b2(g[1]),
                     b2(beta[1]), Wn[2], Ws[2])
    (acc3,) = _sc_agg(y3, row, col, z_h, z_d, ones)
    out = _tc_final(s3, acc3, inv, b2(bs[2]), b2(bn[2]), b2(g[2]),
                    b2(beta[2]), params["Hw1"], b2(params["Hb1"]),
                    params["Hw2"], b2(params["Hb2"]))
    return out.reshape(N)
